# baseline (speedup 1.0000x reference)
import functools
import math

import jax
import jax.numpy as jnp
from jax import lax
from jax.experimental import pallas as pl
from jax.experimental.pallas import tpu as pltpu
from jax.experimental.pallas import tpu_sc as plsc

D_MODEL = 64
SCALE = math.sqrt(D_MODEL)
LANES = 16
NUM_CORES = 2
NUM_SUBCORES = 16
NUM_WORKERS = NUM_CORES * NUM_SUBCORES
CHUNK = 128
NBUF = 4
N_TOK = 200
N_SEQ = 4096
SBLK = N_SEQ // CHUNK


def _emb_body(idx_hbm, tab_hbm, out_hbm, idx_v, rowbufs, obufs, gsems, ssems,
              *, units_per_w):
  wid = lax.axis_index("s") * NUM_CORES + lax.axis_index("c")
  base_u = wid * units_per_w

  pltpu.sync_copy(idx_hbm.at[pl.ds(base_u * CHUNK, units_per_w * CHUNK)],
                  idx_v)

  iota = lax.iota(jnp.int32, LANES)
  tbase = [(q * LANES + iota) * CHUNK for q in range(D_MODEL // LANES)]

  def start_gather(k, b):
    pltpu.async_copy(tab_hbm.at[idx_v.at[pl.ds(k * CHUNK, CHUNK)]],
                     rowbufs[b], gsems[b])

  def wait_gather(b):
    pltpu.make_async_copy(tab_hbm.at[idx_v.at[pl.ds(0, CHUNK)]],
                          rowbufs[b], gsems[b]).wait()

  def transpose_scale(b):
    rows = rowbufs[b]
    ob = obufs[b]

    def ss_body(ss, _):
      for q in range(D_MODEL // LANES):
        v = rows[ss, pl.ds(q * LANES, LANES)] * SCALE
        plsc.store_scatter(ob, [tbase[q] + ss], v)
      return 0

    lax.fori_loop(0, CHUNK, ss_body, 0)

  def start_scatter(k, b):
    u = base_u + k
    t = u // SBLK
    sb = u % SBLK
    for dt in range(D_MODEL // 8):
      off = ((t * 8 + dt) * SBLK + sb) * 1024
      pltpu.async_copy(obufs[b].at[pl.ds(dt * 1024, 1024)],
                       out_hbm.at[pl.ds(off, 1024)], ssems[b])

  def wait_scatter(b):
    for _ in range(D_MODEL // 8):
      pltpu.make_async_copy(obufs[b].at[pl.ds(0, 1024)],
                            out_hbm.at[pl.ds(0, 1024)], ssems[b]).wait()

  for b in range(NBUF):
    start_gather(b, b)

  def step(gg, _):
    for b in range(NBUF):
      k = gg * NBUF + b
      wait_gather(b)
      transpose_scale(b)
      start_scatter(k, b)
    for b in range(NBUF):
      wait_scatter(b)
      start_gather((gg + 1) * NBUF + b, b)
    return 0

  lax.fori_loop(0, units_per_w // NBUF - 1, step, 0)

  for b in range(NBUF):
    k = units_per_w - NBUF + b
    wait_gather(b)
    transpose_scale(b)
    start_scatter(k, b)
    wait_scatter(b)


@jax.jit
def _emb_lookup(idx_flat, lut):
  n = idx_flat.shape[0]
  units = n // CHUNK
  assert units % (NUM_WORKERS * NBUF) == 0
  units_per_w = units // NUM_WORKERS
  mesh = plsc.VectorSubcoreMesh(
      core_axis_name="c", subcore_axis_name="s",
      num_cores=NUM_CORES, num_subcores=NUM_SUBCORES)
  body = functools.partial(_emb_body, units_per_w=units_per_w)
  return pl.kernel(
      body,
      out_type=jax.ShapeDtypeStruct((n * D_MODEL,), jnp.float32),
      mesh=mesh,
      scratch_types=[
          pltpu.VMEM((units_per_w * CHUNK,), jnp.int32),
          [pltpu.VMEM((CHUNK, D_MODEL), jnp.float32) for _ in range(NBUF)],
          [pltpu.VMEM((CHUNK * D_MODEL,), jnp.float32) for _ in range(NBUF)],
          [pltpu.SemaphoreType.DMA for _ in range(NBUF)],
          [pltpu.SemaphoreType.DMA for _ in range(NBUF)],
      ],
      compiler_params=pltpu.CompilerParams(
          use_tc_tiling_on_sc=False, needs_layout_passes=False),
      name="sc_embedding_lookup",
  )(idx_flat, lut)


def kernel(x, lut):
  idx_flat = x.T.reshape(-1).astype(jnp.int32)
  flat = _emb_lookup(idx_flat, lut)
  out5 = flat.reshape(N_TOK, 8, SBLK, 8, CHUNK)
  return out5.transpose(2, 4, 0, 1, 3).reshape(N_SEQ, N_TOK, D_MODEL)


# speedup vs baseline: 1.4975x; 1.4975x over previous
import functools
import math

import jax
import jax.numpy as jnp
from jax import lax
from jax.experimental import pallas as pl
from jax.experimental.pallas import tpu as pltpu
from jax.experimental.pallas import tpu_sc as plsc

D_MODEL = 64
SCALE = math.sqrt(D_MODEL)
LANES = 16
NUM_CORES = 2
NUM_SUBCORES = 16
NUM_WORKERS = NUM_CORES * NUM_SUBCORES
CHUNK = 128
NBUF = 4
N_TOK = 200
N_SEQ = 4096
SBLK = N_SEQ // CHUNK


def _emb_body(idx_hbm, tab_hbm, out_hbm, idx_v, rowbufs, obufs, gsems, ssems,
              *, units_per_w):
  wid = lax.axis_index("s") * NUM_CORES + lax.axis_index("c")
  base_u = wid * units_per_w

  pltpu.sync_copy(idx_hbm.at[pl.ds(base_u * CHUNK, units_per_w * CHUNK)],
                  idx_v)

  iota = lax.iota(jnp.int32, LANES)

  def start_gather(k, b):
    pltpu.async_copy(tab_hbm.at[idx_v.at[pl.ds(k * CHUNK, CHUNK)]],
                     rowbufs[b], gsems[b])

  def wait_gather(b):
    pltpu.make_async_copy(tab_hbm.at[idx_v.at[pl.ds(0, CHUNK)]],
                          rowbufs[b], gsems[b]).wait()

  def transpose_scale(b):
    rows = rowbufs[b]
    ob = obufs[b]

    def blk_body(blk, _):
      ss0 = (blk // 4) * LANES
      q16 = (blk % 4) * LANES
      svec = iota + ss0
      for k in range(LANES):
        mk = (iota + k) & (LANES - 1)
        dvec = mk + q16
        v = plsc.load_gather(rows, [svec, dvec]) * SCALE
        plsc.store_scatter(ob, [dvec, svec], v)
      return 0

    lax.fori_loop(0, (CHUNK // LANES) * (D_MODEL // LANES), blk_body, 0)

  def start_scatter(k, b):
    u = base_u + k
    t = u // SBLK
    sb = u % SBLK
    for dt in range(D_MODEL // 8):
      r0 = ((t * 8 + dt) * SBLK + sb) * 8
      pltpu.async_copy(obufs[b].at[pl.ds(dt * 8, 8), :],
                       out_hbm.at[pl.ds(r0, 8), :], ssems[b])

  def wait_scatter(b):
    for _ in range(D_MODEL // 8):
      pltpu.make_async_copy(obufs[b].at[pl.ds(0, 8), :],
                            out_hbm.at[pl.ds(0, 8), :], ssems[b]).wait()

  for b in range(NBUF):
    start_gather(b, b)

  def step(gg, _):
    for b in range(NBUF):
      k = gg * NBUF + b
      wait_gather(b)
      transpose_scale(b)
      start_scatter(k, b)
    for b in range(NBUF):
      wait_scatter(b)
      start_gather((gg + 1) * NBUF + b, b)
    return 0

  lax.fori_loop(0, units_per_w // NBUF - 1, step, 0)

  for b in range(NBUF):
    k = units_per_w - NBUF + b
    wait_gather(b)
    transpose_scale(b)
    start_scatter(k, b)
    wait_scatter(b)


@jax.jit
def _emb_lookup(idx_flat, lut):
  n = idx_flat.shape[0]
  units = n // CHUNK
  assert units % (NUM_WORKERS * NBUF) == 0
  units_per_w = units // NUM_WORKERS
  mesh = plsc.VectorSubcoreMesh(
      core_axis_name="c", subcore_axis_name="s",
      num_cores=NUM_CORES, num_subcores=NUM_SUBCORES)
  body = functools.partial(_emb_body, units_per_w=units_per_w)
  return pl.kernel(
      body,
      out_type=jax.ShapeDtypeStruct((n * D_MODEL // CHUNK, CHUNK),
                                    jnp.float32),
      mesh=mesh,
      scratch_types=[
          pltpu.VMEM((units_per_w * CHUNK,), jnp.int32),
          [pltpu.VMEM((CHUNK, D_MODEL), jnp.float32) for _ in range(NBUF)],
          [pltpu.VMEM((D_MODEL, CHUNK), jnp.float32) for _ in range(NBUF)],
          [pltpu.SemaphoreType.DMA for _ in range(NBUF)],
          [pltpu.SemaphoreType.DMA for _ in range(NBUF)],
      ],
      compiler_params=pltpu.CompilerParams(
          use_tc_tiling_on_sc=False, needs_layout_passes=False),
      name="sc_embedding_lookup",
  )(idx_flat, lut)


def kernel(x, lut):
  idx_flat = x.T.reshape(-1).astype(jnp.int32)
  out2 = _emb_lookup(idx_flat, lut)
  out5 = out2.reshape(N_TOK, 8, SBLK, 8, CHUNK)
  return out5.transpose(2, 4, 0, 1, 3).reshape(N_SEQ, N_TOK, D_MODEL)


# speedup vs baseline: 1.8649x; 1.2453x over previous
import functools
import math

import jax
import jax.numpy as jnp
from jax import lax
from jax.experimental import pallas as pl
from jax.experimental.pallas import tpu as pltpu
from jax.experimental.pallas import tpu_sc as plsc

D_MODEL = 64
SCALE = math.sqrt(D_MODEL)
LANES = 16
NUM_CORES = 2
NUM_SUBCORES = 16
NUM_WORKERS = NUM_CORES * NUM_SUBCORES
CHUNK = 128
NBUF = 4
N_TOK = 200
N_SEQ = 4096
SBLK = N_SEQ // CHUNK
VOCAB = 1000000
VFULL = VOCAB // CHUNK
VREM = VOCAB - VFULL * CHUNK
TP_PER_W = VFULL // NUM_WORKERS
TP_EXTRA = VFULL - TP_PER_W * NUM_WORKERS


def _diag_transpose(iota, src, dst, nr, nc, scale=None):
  def blk_body(blk, _):
    r0 = (blk // nc) * LANES
    c0 = (blk % nc) * LANES
    rvec = iota + r0

    @plsc.parallel_loop(0, LANES, unroll=8)
    def _k(k):
      mk = (iota + k) & (LANES - 1)
      cvec = mk + c0
      v = plsc.load_gather(src, [rvec, cvec])
      if scale is not None:
        v = v * scale
      plsc.store_scatter(dst, [cvec, rvec], v)

    return 0

  lax.fori_loop(0, nr * nc, blk_body, 0)


def _tp_body(src_hbm, rem_hbm, dst_hbm, ibufs, obufs, isems, osems):
  wid = lax.axis_index("s") * NUM_CORES + lax.axis_index("c")
  iota = lax.iota(jnp.int32, LANES)
  base = wid * TP_PER_W

  def start_in(vc, b, width=CHUNK):
    for dr in range(D_MODEL // 8):
      pltpu.async_copy(
          src_hbm.at[pl.ds(dr * 8, 8), pl.ds(vc * CHUNK, width)],
          ibufs[b].at[pl.ds(dr * 8, 8), pl.ds(0, width)], isems[b])

  def wait_in(b, width=CHUNK):
    for _ in range(D_MODEL // 8):
      pltpu.make_async_copy(
          src_hbm.at[pl.ds(0, 8), pl.ds(0, width)],
          ibufs[b].at[pl.ds(0, 8), pl.ds(0, width)], isems[b]).wait()

  def start_out(vc, b, width=CHUNK):
    pltpu.async_copy(obufs[b].at[pl.ds(0, width), :],
                     dst_hbm.at[pl.ds(vc * CHUNK, width), :], osems[b])

  def wait_out(b, width=CHUNK):
    pltpu.make_async_copy(obufs[b].at[pl.ds(0, width), :],
                          dst_hbm.at[pl.ds(0, width), :], osems[b]).wait()

  for b in range(NBUF):
    start_in(base + b, b)

  for b in range(NBUF):
    wait_in(b)
    _diag_transpose(iota, ibufs[b], obufs[b],
                    D_MODEL // LANES, CHUNK // LANES)
    start_out(base + b, b)
    start_in(base + NBUF + b, b)

  def step(gg, _):
    for b in range(NBUF):
      wait_in(b)
      wait_out(b)
      _diag_transpose(iota, ibufs[b], obufs[b],
                      D_MODEL // LANES, CHUNK // LANES)
      start_out(base + gg * NBUF + b, b)
      start_in(base + (gg + 1) * NBUF + b, b)
    return 0

  lax.fori_loop(1, TP_PER_W // NBUF - 1, step, 0)

  for b in range(NBUF):
    wait_in(b)
    wait_out(b)
    _diag_transpose(iota, ibufs[b], obufs[b],
                    D_MODEL // LANES, CHUNK // LANES)
    start_out(base + TP_PER_W - NBUF + b, b)
    wait_out(b)

  @pl.when(wid < TP_EXTRA)
  def _extra_full():
    vc = VFULL - TP_EXTRA + wid
    start_in(vc, 0)
    wait_in(0)
    _diag_transpose(iota, ibufs[0], obufs[0],
                    D_MODEL // LANES, CHUNK // LANES)
    start_out(vc, 0)
    wait_out(0)

  @pl.when(wid == TP_EXTRA)
  def _rem():
    pltpu.async_copy(rem_hbm, obufs[1].at[pl.ds(0, VREM), :],
                     osems[1]).wait()
    pltpu.async_copy(obufs[1].at[pl.ds(0, VREM), :],
                     dst_hbm.at[pl.ds(VFULL * CHUNK, VREM), :],
                     osems[1]).wait()


@jax.jit
def _transpose_table(lut_t, rem_rm):
  mesh = plsc.VectorSubcoreMesh(
      core_axis_name="c", subcore_axis_name="s",
      num_cores=NUM_CORES, num_subcores=NUM_SUBCORES)
  return pl.kernel(
      _tp_body,
      out_type=jax.ShapeDtypeStruct((VOCAB, D_MODEL), jnp.float32),
      mesh=mesh,
      scratch_types=[
          [pltpu.VMEM((D_MODEL, CHUNK), jnp.float32) for _ in range(NBUF)],
          [pltpu.VMEM((CHUNK, D_MODEL), jnp.float32) for _ in range(NBUF)],
          [pltpu.SemaphoreType.DMA for _ in range(NBUF)],
          [pltpu.SemaphoreType.DMA for _ in range(NBUF)],
      ],
      compiler_params=pltpu.CompilerParams(needs_layout_passes=False),
      name="sc_table_relayout",
  )(lut_t, rem_rm)


def _emb_body(idx_hbm, tab_hbm, out_hbm, idx_v, rowbufs, obufs, gsems, ssems,
              *, units_per_w):
  wid = lax.axis_index("s") * NUM_CORES + lax.axis_index("c")
  base_u = wid * units_per_w

  pltpu.sync_copy(idx_hbm.at[pl.ds(base_u * CHUNK, units_per_w * CHUNK)],
                  idx_v)

  iota = lax.iota(jnp.int32, LANES)

  def start_gather(k, b):
    pltpu.async_copy(tab_hbm.at[idx_v.at[pl.ds(k * CHUNK, CHUNK)]],
                     rowbufs[b], gsems[b])

  def wait_gather(b):
    pltpu.make_async_copy(tab_hbm.at[idx_v.at[pl.ds(0, CHUNK)]],
                          rowbufs[b], gsems[b]).wait()

  def start_scatter(k, b):
    u = base_u + k
    t = u // SBLK
    sb = u % SBLK
    for dt in range(D_MODEL // 8):
      r0 = ((t * 8 + dt) * SBLK + sb) * 8
      pltpu.async_copy(obufs[b].at[pl.ds(dt * 8, 8), :],
                       out_hbm.at[pl.ds(r0, 8), :], ssems[b])

  def wait_scatter(b):
    for _ in range(D_MODEL // 8):
      pltpu.make_async_copy(obufs[b].at[pl.ds(0, 8), :],
                            out_hbm.at[pl.ds(0, 8), :], ssems[b]).wait()

  for b in range(NBUF):
    start_gather(b, b)

  for b in range(NBUF):
    wait_gather(b)
    _diag_transpose(iota, rowbufs[b], obufs[b],
                    CHUNK // LANES, D_MODEL // LANES, scale=SCALE)
    start_scatter(b, b)
    start_gather(NBUF + b, b)

  def step(gg, _):
    for b in range(NBUF):
      k = gg * NBUF + b
      wait_gather(b)
      wait_scatter(b)
      _diag_transpose(iota, rowbufs[b], obufs[b],
                      CHUNK // LANES, D_MODEL // LANES, scale=SCALE)
      start_scatter(k, b)
      start_gather(k + NBUF, b)
    return 0

  lax.fori_loop(1, units_per_w // NBUF - 1, step, 0)

  for b in range(NBUF):
    k = units_per_w - NBUF + b
    wait_gather(b)
    wait_scatter(b)
    _diag_transpose(iota, rowbufs[b], obufs[b],
                    CHUNK // LANES, D_MODEL // LANES, scale=SCALE)
    start_scatter(k, b)
    wait_scatter(b)


@jax.jit
def _emb_lookup(idx_flat, lut_rm):
  n = idx_flat.shape[0]
  units = n // CHUNK
  assert units % (NUM_WORKERS * NBUF) == 0
  units_per_w = units // NUM_WORKERS
  mesh = plsc.VectorSubcoreMesh(
      core_axis_name="c", subcore_axis_name="s",
      num_cores=NUM_CORES, num_subcores=NUM_SUBCORES)
  body = functools.partial(_emb_body, units_per_w=units_per_w)
  return pl.kernel(
      body,
      out_type=jax.ShapeDtypeStruct((n * D_MODEL // CHUNK, CHUNK),
                                    jnp.float32),
      mesh=mesh,
      scratch_types=[
          pltpu.VMEM((units_per_w * CHUNK,), jnp.int32),
          [pltpu.VMEM((CHUNK, D_MODEL), jnp.float32) for _ in range(NBUF)],
          [pltpu.VMEM((D_MODEL, CHUNK), jnp.float32) for _ in range(NBUF)],
          [pltpu.SemaphoreType.DMA for _ in range(NBUF)],
          [pltpu.SemaphoreType.DMA for _ in range(NBUF)],
      ],
      compiler_params=pltpu.CompilerParams(
          use_tc_tiling_on_sc=False, needs_layout_passes=False),
      name="sc_embedding_lookup",
  )(idx_flat, lut_rm)


def kernel(x, lut):
  idx_flat = x.T.reshape(-1).astype(jnp.int32)
  rem_rm = lut[VFULL * CHUNK:, :]
  lut_rm = _transpose_table(lut.T, rem_rm)
  out2 = _emb_lookup(idx_flat, lut_rm)
  out5 = out2.reshape(N_TOK, 8, SBLK, 8, CHUNK)
  return out5.transpose(2, 4, 0, 1, 3).reshape(N_SEQ, N_TOK, D_MODEL)


# speedup vs baseline: 3.0389x; 1.6295x over previous
import functools
import math

import jax
import jax.numpy as jnp
from jax import lax
from jax.experimental import pallas as pl
from jax.experimental.pallas import tpu as pltpu
from jax.experimental.pallas import tpu_sc as plsc

D_MODEL = 64
SCALE = math.sqrt(D_MODEL)
LANES = 16
NUM_CORES = 2
NUM_SUBCORES = 16
NUM_WORKERS = NUM_CORES * NUM_SUBCORES
CHUNK = 128
NBUF = 4
N_TOK = 200
N_SEQ = 4096
SBLK = N_SEQ // CHUNK
VOCAB = 1000000
VFULL = VOCAB // CHUNK
VREM = VOCAB - VFULL * CHUNK
TP_PER_W = VFULL // NUM_WORKERS
TP_EXTRA = VFULL - TP_PER_W * NUM_WORKERS


def _diag_transpose(iota, src, dst, nr, nc, scale=None):
  rlen = LANES * nr

  def blk_body(blk, _):
    r0 = (blk // nc) * LANES
    c0 = (blk % nc) * LANES
    rvec = iota + r0

    @plsc.parallel_loop(0, LANES, unroll=8)
    def _k(k):
      mk = (iota + k) & (LANES - 1)
      cvec = mk + c0
      v = plsc.load_gather(src, [rvec, cvec])
      if scale is not None:
        v = v * scale
      plsc.store_scatter(dst, [cvec * rlen + rvec], v)

    return 0

  lax.fori_loop(0, nr * nc, blk_body, 0)


def _tp_body(src_hbm, rem_hbm, dst_hbm, ibufs, obufs, isems, osems):
  wid = lax.axis_index("s") * NUM_CORES + lax.axis_index("c")
  iota = lax.iota(jnp.int32, LANES)
  base = wid * TP_PER_W

  def start_in(vc, b, width=CHUNK):
    for dr in range(D_MODEL // 8):
      pltpu.async_copy(
          src_hbm.at[pl.ds(dr * 8, 8), pl.ds(vc * CHUNK, width)],
          ibufs[b].at[pl.ds(dr * 8, 8), pl.ds(0, width)], isems[b])

  def wait_in(b, width=CHUNK):
    for _ in range(D_MODEL // 8):
      pltpu.make_async_copy(
          src_hbm.at[pl.ds(0, 8), pl.ds(0, width)],
          ibufs[b].at[pl.ds(0, 8), pl.ds(0, width)], isems[b]).wait()

  def start_out(vc, b, width=CHUNK):
    pltpu.async_copy(obufs[b].at[pl.ds(0, width * D_MODEL)],
                     dst_hbm.at[pl.ds(vc * CHUNK * D_MODEL, width * D_MODEL)],
                     osems[b])

  def wait_out(b, width=CHUNK):
    pltpu.make_async_copy(obufs[b].at[pl.ds(0, width * D_MODEL)],
                          dst_hbm.at[pl.ds(0, width * D_MODEL)],
                          osems[b]).wait()

  for b in range(NBUF):
    start_in(base + b, b)

  for b in range(NBUF):
    wait_in(b)
    _diag_transpose(iota, ibufs[b], obufs[b],
                    D_MODEL // LANES, CHUNK // LANES)
    start_out(base + b, b)
    start_in(base + NBUF + b, b)

  def step(gg, _):
    for b in range(NBUF):
      wait_in(b)
      wait_out(b)
      _diag_transpose(iota, ibufs[b], obufs[b],
                      D_MODEL // LANES, CHUNK // LANES)
      start_out(base + gg * NBUF + b, b)
      start_in(base + (gg + 1) * NBUF + b, b)
    return 0

  lax.fori_loop(1, TP_PER_W // NBUF - 1, step, 0)

  for b in range(NBUF):
    wait_in(b)
    wait_out(b)
    _diag_transpose(iota, ibufs[b], obufs[b],
                    D_MODEL // LANES, CHUNK // LANES)
    start_out(base + TP_PER_W - NBUF + b, b)
    wait_out(b)

  @pl.when(wid < TP_EXTRA)
  def _extra_full():
    vc = VFULL - TP_EXTRA + wid
    start_in(vc, 0)
    wait_in(0)
    _diag_transpose(iota, ibufs[0], obufs[0],
                    D_MODEL // LANES, CHUNK // LANES)
    start_out(vc, 0)
    wait_out(0)

  @pl.when(wid == TP_EXTRA)
  def _rem():
    pltpu.async_copy(rem_hbm, obufs[1].at[pl.ds(0, VREM * D_MODEL)],
                     osems[1]).wait()
    pltpu.async_copy(obufs[1].at[pl.ds(0, VREM * D_MODEL)],
                     dst_hbm.at[pl.ds(VFULL * CHUNK * D_MODEL,
                                      VREM * D_MODEL)],
                     osems[1]).wait()


@jax.jit
def _transpose_table(lut_t, rem_rm):
  mesh = plsc.VectorSubcoreMesh(
      core_axis_name="c", subcore_axis_name="s",
      num_cores=NUM_CORES, num_subcores=NUM_SUBCORES)
  return pl.kernel(
      _tp_body,
      out_type=jax.ShapeDtypeStruct((VOCAB * D_MODEL,), jnp.float32),
      mesh=mesh,
      scratch_types=[
          [pltpu.VMEM((D_MODEL, CHUNK), jnp.float32) for _ in range(NBUF)],
          [pltpu.VMEM((CHUNK * D_MODEL,), jnp.float32) for _ in range(NBUF)],
          [pltpu.SemaphoreType.DMA for _ in range(NBUF)],
          [pltpu.SemaphoreType.DMA for _ in range(NBUF)],
      ],
      compiler_params=pltpu.CompilerParams(needs_layout_passes=False),
      name="sc_table_relayout",
  )(lut_t, rem_rm)


def _emb_body(idx_hbm, tab_hbm, out_hbm, idx_v, rowbufs, obufs, gsems, ssems,
              *, units_per_w):
  wid = lax.axis_index("s") * NUM_CORES + lax.axis_index("c")
  base_u = wid * units_per_w

  pltpu.sync_copy(idx_hbm.at[pl.ds(base_u * CHUNK, units_per_w * CHUNK)],
                  idx_v)

  iota = lax.iota(jnp.int32, LANES)

  def start_gather(k, b):
    pltpu.async_copy(tab_hbm.at[idx_v.at[pl.ds(k * CHUNK, CHUNK)]],
                     rowbufs[b], gsems[b])

  def wait_gather(b):
    pltpu.make_async_copy(tab_hbm.at[idx_v.at[pl.ds(0, CHUNK)]],
                          rowbufs[b], gsems[b]).wait()

  def start_scatter(k, b):
    u = base_u + k
    t = u // SBLK
    sb = u % SBLK
    for dt in range(D_MODEL // 8):
      off = (((t * 8 + dt) * SBLK + sb) * 8) * CHUNK
      pltpu.async_copy(obufs[b].at[pl.ds(dt * 8 * CHUNK, 8 * CHUNK)],
                       out_hbm.at[pl.ds(off, 8 * CHUNK)], ssems[b])

  def wait_scatter(b):
    for _ in range(D_MODEL // 8):
      pltpu.make_async_copy(obufs[b].at[pl.ds(0, 8 * CHUNK)],
                            out_hbm.at[pl.ds(0, 8 * CHUNK)], ssems[b]).wait()

  for b in range(NBUF):
    start_gather(b, b)

  for b in range(NBUF):
    wait_gather(b)
    _diag_transpose(iota, rowbufs[b], obufs[b],
                    CHUNK // LANES, D_MODEL // LANES, scale=SCALE)
    start_scatter(b, b)
    start_gather(NBUF + b, b)

  def step(gg, _):
    for b in range(NBUF):
      k = gg * NBUF + b
      wait_gather(b)
      wait_scatter(b)
      _diag_transpose(iota, rowbufs[b], obufs[b],
                      CHUNK // LANES, D_MODEL // LANES, scale=SCALE)
      start_scatter(k, b)
      start_gather(k + NBUF, b)
    return 0

  lax.fori_loop(1, units_per_w // NBUF - 1, step, 0)

  for b in range(NBUF):
    k = units_per_w - NBUF + b
    wait_gather(b)
    wait_scatter(b)
    _diag_transpose(iota, rowbufs[b], obufs[b],
                    CHUNK // LANES, D_MODEL // LANES, scale=SCALE)
    start_scatter(k, b)
    wait_scatter(b)


@jax.jit
def _emb_lookup(idx_flat, lut_rm):
  n = idx_flat.shape[0]
  units = n // CHUNK
  assert units % (NUM_WORKERS * NBUF) == 0
  units_per_w = units // NUM_WORKERS
  mesh = plsc.VectorSubcoreMesh(
      core_axis_name="c", subcore_axis_name="s",
      num_cores=NUM_CORES, num_subcores=NUM_SUBCORES)
  body = functools.partial(_emb_body, units_per_w=units_per_w)
  return pl.kernel(
      body,
      out_type=jax.ShapeDtypeStruct((n * D_MODEL,), jnp.float32),
      mesh=mesh,
      scratch_types=[
          pltpu.VMEM((units_per_w * CHUNK,), jnp.int32),
          [pltpu.VMEM((CHUNK, D_MODEL), jnp.float32) for _ in range(NBUF)],
          [pltpu.VMEM((CHUNK * D_MODEL,), jnp.float32) for _ in range(NBUF)],
          [pltpu.SemaphoreType.DMA for _ in range(NBUF)],
          [pltpu.SemaphoreType.DMA for _ in range(NBUF)],
      ],
      compiler_params=pltpu.CompilerParams(
          use_tc_tiling_on_sc=False, needs_layout_passes=False),
      name="sc_embedding_lookup",
  )(idx_flat, lut_rm)


def kernel(x, lut):
  idx_flat = x.T.reshape(-1).astype(jnp.int32)
  rem_rm = lut[VFULL * CHUNK:, :].reshape(-1)
  lut_rm = _transpose_table(lut.T, rem_rm).reshape(VOCAB, D_MODEL)
  flat = _emb_lookup(idx_flat, lut_rm)
  out5 = flat.reshape(N_TOK, 8, SBLK, 8, CHUNK)
  return out5.transpose(2, 4, 0, 1, 3).reshape(N_SEQ, N_TOK, D_MODEL)


# speedup vs baseline: 3.3732x; 1.1100x over previous
import functools
import math

import jax
import jax.numpy as jnp
from jax import lax
from jax.experimental import pallas as pl
from jax.experimental.pallas import tpu as pltpu
from jax.experimental.pallas import tpu_sc as plsc

D_MODEL = 64
SCALE = math.sqrt(D_MODEL)
LANES = 16
NUM_CORES = 2
NUM_SUBCORES = 16
NUM_WORKERS = NUM_CORES * NUM_SUBCORES
CHUNK = 128
NBUF = 4
N_TOK = 200
N_SEQ = 4096
SBLK = N_SEQ // CHUNK
VOCAB = 1000000
VFULL = VOCAB // CHUNK
VREM = VOCAB - VFULL * CHUNK
TP_PER_W = VFULL // NUM_WORKERS
TP_EXTRA = VFULL - TP_PER_W * NUM_WORKERS


def _diag_transpose_flat(iota, src_flat, dst_flat, nr, nc, scale=None):
  clen = LANES * nc
  rlen = LANES * nr
  iota_c = iota * clen

  @plsc.parallel_loop(0, nr * nc, unroll=1)
  def blk_body(blk):
    r0 = (blk // nc) * LANES
    c0 = (blk % nc) * LANES
    sg = r0 * clen + c0
    sd = c0 * rlen + r0

    @plsc.parallel_loop(0, LANES, unroll=8)
    def _k(k):
      mk = (iota + k) & (LANES - 1)
      gidx = iota_c + mk + sg
      v = plsc.load_gather(src_flat, [gidx])
      if scale is not None:
        v = v * scale
      sidx = mk * rlen + iota + sd
      plsc.store_scatter(dst_flat, [sidx], v)


def _diag_transpose_2d(iota, src, dst_flat, nr, nc, scale=None):
  rlen = LANES * nr

  @plsc.parallel_loop(0, nr * nc, unroll=1)
  def blk_body(blk):
    r0 = (blk // nc) * LANES
    c0 = (blk % nc) * LANES
    rvec = iota + r0
    sd = c0 * rlen + r0

    @plsc.parallel_loop(0, LANES, unroll=8)
    def _k(k):
      mk = (iota + k) & (LANES - 1)
      v = plsc.load_gather(src, [rvec, mk + c0])
      if scale is not None:
        v = v * scale
      plsc.store_scatter(dst_flat, [mk * rlen + iota + sd], v)


def _tp_body(src_hbm, rem_hbm, dst_hbm, ibufs, obufs, isems, osems):
  wid = lax.axis_index("s") * NUM_CORES + lax.axis_index("c")
  iota = lax.iota(jnp.int32, LANES)
  base = wid * TP_PER_W

  def tp(b):
    _diag_transpose_2d(iota, ibufs[b], obufs[b],
                       D_MODEL // LANES, CHUNK // LANES)

  def start_in(vc, b):
    for dr in range(D_MODEL // 8):
      pltpu.async_copy(
          src_hbm.at[pl.ds(dr * 8, 8), pl.ds(vc * CHUNK, CHUNK)],
          ibufs[b].at[pl.ds(dr * 8, 8), :], isems[b])

  def wait_in(b):
    for _ in range(D_MODEL // 8):
      pltpu.make_async_copy(
          src_hbm.at[pl.ds(0, 8), pl.ds(0, CHUNK)],
          ibufs[b].at[pl.ds(0, 8), :], isems[b]).wait()

  def start_out(vc, b, width=CHUNK):
    pltpu.async_copy(obufs[b].at[pl.ds(0, width * D_MODEL)],
                     dst_hbm.at[pl.ds(vc * CHUNK * D_MODEL, width * D_MODEL)],
                     osems[b])

  def wait_out(b, width=CHUNK):
    pltpu.make_async_copy(obufs[b].at[pl.ds(0, width * D_MODEL)],
                          dst_hbm.at[pl.ds(0, width * D_MODEL)],
                          osems[b]).wait()

  for b in range(NBUF):
    start_in(base + b, b)

  for b in range(NBUF):
    wait_in(b)
    tp(b)
    start_out(base + b, b)
    start_in(base + NBUF + b, b)

  def step(gg, _):
    for b in range(NBUF):
      wait_in(b)
      wait_out(b)
      tp(b)
      start_out(base + gg * NBUF + b, b)
      start_in(base + (gg + 1) * NBUF + b, b)
    return 0

  lax.fori_loop(1, TP_PER_W // NBUF - 1, step, 0)

  for b in range(NBUF):
    wait_in(b)
    wait_out(b)
    tp(b)
    start_out(base + TP_PER_W - NBUF + b, b)
    wait_out(b)

  @pl.when(wid < TP_EXTRA)
  def _extra_full():
    vc = VFULL - TP_EXTRA + wid
    start_in(vc, 0)
    wait_in(0)
    tp(0)
    start_out(vc, 0)
    wait_out(0)

  @pl.when(wid == TP_EXTRA)
  def _rem():
    pltpu.async_copy(rem_hbm, obufs[1].at[pl.ds(0, VREM * D_MODEL)],
                     osems[1]).wait()
    pltpu.async_copy(obufs[1].at[pl.ds(0, VREM * D_MODEL)],
                     dst_hbm.at[pl.ds(VFULL * CHUNK * D_MODEL,
                                      VREM * D_MODEL)],
                     osems[1]).wait()


@jax.jit
def _transpose_table(lut_t, rem_rm):
  mesh = plsc.VectorSubcoreMesh(
      core_axis_name="c", subcore_axis_name="s",
      num_cores=NUM_CORES, num_subcores=NUM_SUBCORES)
  return pl.kernel(
      _tp_body,
      out_type=jax.ShapeDtypeStruct((VOCAB * D_MODEL,), jnp.float32),
      mesh=mesh,
      scratch_types=[
          [pltpu.VMEM((D_MODEL, CHUNK), jnp.float32) for _ in range(NBUF)],
          [pltpu.VMEM((CHUNK * D_MODEL,), jnp.float32) for _ in range(NBUF)],
          [pltpu.SemaphoreType.DMA for _ in range(NBUF)],
          [pltpu.SemaphoreType.DMA for _ in range(NBUF)],
      ],
      compiler_params=pltpu.CompilerParams(needs_layout_passes=False),
      name="sc_table_relayout",
  )(lut_t, rem_rm)


def _emb_body(idx_hbm, tab_hbm, out_hbm, idx_v, rowbufs, obufs, gsems, ssems,
              *, units_per_w):
  wid = lax.axis_index("s") * NUM_CORES + lax.axis_index("c")
  base_u = wid * units_per_w

  pltpu.sync_copy(idx_hbm.at[pl.ds(base_u * CHUNK, units_per_w * CHUNK)],
                  idx_v)

  iota = lax.iota(jnp.int32, LANES)

  def tp(b):
    _diag_transpose_2d(iota, rowbufs[b], obufs[b],
                       CHUNK // LANES, D_MODEL // LANES, scale=SCALE)

  def start_gather(k, b):
    pltpu.async_copy(tab_hbm.at[idx_v.at[pl.ds(k * CHUNK, CHUNK)]],
                     rowbufs[b], gsems[b])

  def wait_gather(b):
    pltpu.make_async_copy(tab_hbm.at[idx_v.at[pl.ds(0, CHUNK)]],
                          rowbufs[b], gsems[b]).wait()

  def start_scatter(k, b):
    u = base_u + k
    t = u // SBLK
    sb = u % SBLK
    for dt in range(D_MODEL // 8):
      off = (((t * 8 + dt) * SBLK + sb) * 8) * CHUNK
      pltpu.async_copy(obufs[b].at[pl.ds(dt * 8 * CHUNK, 8 * CHUNK)],
                       out_hbm.at[pl.ds(off, 8 * CHUNK)], ssems[b])

  def wait_scatter(b):
    for _ in range(D_MODEL // 8):
      pltpu.make_async_copy(obufs[b].at[pl.ds(0, 8 * CHUNK)],
                            out_hbm.at[pl.ds(0, 8 * CHUNK)], ssems[b]).wait()

  for b in range(NBUF):
    start_gather(b, b)

  for b in range(NBUF):
    wait_gather(b)
    tp(b)
    start_scatter(b, b)
    start_gather(NBUF + b, b)

  def step(gg, _):
    for b in range(NBUF):
      k = gg * NBUF + b
      wait_gather(b)
      wait_scatter(b)
      tp(b)
      start_scatter(k, b)
      start_gather(k + NBUF, b)
    return 0

  lax.fori_loop(1, units_per_w // NBUF - 1, step, 0)

  for b in range(NBUF):
    k = units_per_w - NBUF + b
    wait_gather(b)
    wait_scatter(b)
    tp(b)
    start_scatter(k, b)
    wait_scatter(b)


@jax.jit
def _emb_lookup(idx_flat, lut_rm):
  n = idx_flat.shape[0]
  units = n // CHUNK
  assert units % (NUM_WORKERS * NBUF) == 0
  units_per_w = units // NUM_WORKERS
  mesh = plsc.VectorSubcoreMesh(
      core_axis_name="c", subcore_axis_name="s",
      num_cores=NUM_CORES, num_subcores=NUM_SUBCORES)
  body = functools.partial(_emb_body, units_per_w=units_per_w)
  return pl.kernel(
      body,
      out_type=jax.ShapeDtypeStruct((n * D_MODEL,), jnp.float32),
      mesh=mesh,
      scratch_types=[
          pltpu.VMEM((units_per_w * CHUNK,), jnp.int32),
          [pltpu.VMEM((CHUNK, D_MODEL), jnp.float32) for _ in range(NBUF)],
          [pltpu.VMEM((CHUNK * D_MODEL,), jnp.float32) for _ in range(NBUF)],
          [pltpu.SemaphoreType.DMA for _ in range(NBUF)],
          [pltpu.SemaphoreType.DMA for _ in range(NBUF)],
      ],
      compiler_params=pltpu.CompilerParams(
          use_tc_tiling_on_sc=False, needs_layout_passes=False),
      name="sc_embedding_lookup",
  )(idx_flat, lut_rm)


def kernel(x, lut):
  idx_flat = x.T.reshape(-1).astype(jnp.int32)
  rem_rm = lut[VFULL * CHUNK:, :].reshape(-1)
  lut_rm = _transpose_table(lut.T, rem_rm).reshape(VOCAB, D_MODEL)
  flat = _emb_lookup(idx_flat, lut_rm)
  out5 = flat.reshape(N_TOK, 8, SBLK, 8, CHUNK)
  return out5.transpose(2, 4, 0, 1, 3).reshape(N_SEQ, N_TOK, D_MODEL)


# speedup vs baseline: 5.1022x; 1.5126x over previous
import functools
import math

import jax
import jax.numpy as jnp
from jax import lax
from jax.experimental import pallas as pl
from jax.experimental.pallas import tpu as pltpu
from jax.experimental.pallas import tpu_sc as plsc

D_MODEL = 64
SCALE = math.sqrt(D_MODEL)
LANES = 16
NUM_CORES = 2
NUM_SUBCORES = 16
NUM_WORKERS = NUM_CORES * NUM_SUBCORES
CHUNK = 128
NBUF = 4
N_TOK = 200
N_SEQ = 4096
SBLK = N_SEQ // CHUNK
VOCAB = 1000000
VFULL = VOCAB // CHUNK
VREM = VOCAB - VFULL * CHUNK
TP_PER_W = VFULL // NUM_WORKERS
TP_EXTRA = VFULL - TP_PER_W * NUM_WORKERS


def _diag_transpose_flat(iota, src_flat, dst_flat, nr, nc, scale=None):
  clen = LANES * nc
  rlen = LANES * nr
  iota_c = iota * clen

  @plsc.parallel_loop(0, nr * nc, unroll=2)
  def blk_body(blk):
    r0 = (blk // nc) * LANES
    c0 = (blk % nc) * LANES
    sg = r0 * clen + c0
    sd = c0 * rlen + r0

    @plsc.parallel_loop(0, LANES, unroll=16)
    def _k(k):
      mk = (iota + k) & (LANES - 1)
      gidx = iota_c + mk + sg
      v = plsc.load_gather(src_flat, [gidx])
      if scale is not None:
        v = v * scale
      sidx = mk * rlen + iota + sd
      plsc.store_scatter(dst_flat, [sidx], v)


def _diag_transpose_2d(iota, src, dst_flat, nr, nc, scale=None):
  rlen = LANES * nr

  @plsc.parallel_loop(0, nr * nc, unroll=2)
  def blk_body(blk):
    r0 = (blk // nc) * LANES
    c0 = (blk % nc) * LANES
    rvec = iota + r0
    sd = c0 * rlen + r0

    @plsc.parallel_loop(0, LANES, unroll=16)
    def _k(k):
      mk = (iota + k) & (LANES - 1)
      v = plsc.load_gather(src, [rvec, mk + c0])
      if scale is not None:
        v = v * scale
      plsc.store_scatter(dst_flat, [mk * rlen + iota + sd], v)


def _tp_body(src_hbm, rem_hbm, dst_hbm, ibufs, obufs, isems, osems):
  wid = lax.axis_index("s") * NUM_CORES + lax.axis_index("c")
  iota = lax.iota(jnp.int32, LANES)
  base = wid * TP_PER_W

  def tp(b):
    _diag_transpose_2d(iota, ibufs[b], obufs[b],
                       D_MODEL // LANES, CHUNK // LANES)

  def start_in(vc, b):
    for dr in range(D_MODEL // 8):
      pltpu.async_copy(
          src_hbm.at[pl.ds(dr * 8, 8), pl.ds(vc * CHUNK, CHUNK)],
          ibufs[b].at[pl.ds(dr * 8, 8), :], isems[b])

  def wait_in(b):
    for _ in range(D_MODEL // 8):
      pltpu.make_async_copy(
          src_hbm.at[pl.ds(0, 8), pl.ds(0, CHUNK)],
          ibufs[b].at[pl.ds(0, 8), :], isems[b]).wait()

  def start_out(vc, b, width=CHUNK):
    pltpu.async_copy(obufs[b].at[pl.ds(0, width * D_MODEL)],
                     dst_hbm.at[pl.ds(vc * CHUNK * D_MODEL, width * D_MODEL)],
                     osems[b])

  def wait_out(b, width=CHUNK):
    pltpu.make_async_copy(obufs[b].at[pl.ds(0, width * D_MODEL)],
                          dst_hbm.at[pl.ds(0, width * D_MODEL)],
                          osems[b]).wait()

  for b in range(NBUF):
    start_in(base + b, b)

  for b in range(NBUF):
    wait_in(b)
    tp(b)
    start_out(base + b, b)
    start_in(base + NBUF + b, b)

  def step(gg, _):
    for b in range(NBUF):
      wait_in(b)
      wait_out(b)
      tp(b)
      start_out(base + gg * NBUF + b, b)
      start_in(base + (gg + 1) * NBUF + b, b)
    return 0

  lax.fori_loop(1, TP_PER_W // NBUF - 1, step, 0)

  for b in range(NBUF):
    wait_in(b)
    wait_out(b)
    tp(b)
    start_out(base + TP_PER_W - NBUF + b, b)
    wait_out(b)

  @pl.when(wid < TP_EXTRA)
  def _extra_full():
    vc = VFULL - TP_EXTRA + wid
    start_in(vc, 0)
    wait_in(0)
    tp(0)
    start_out(vc, 0)
    wait_out(0)

  @pl.when(wid == TP_EXTRA)
  def _rem():
    pltpu.async_copy(rem_hbm, obufs[1].at[pl.ds(0, VREM * D_MODEL)],
                     osems[1]).wait()
    pltpu.async_copy(obufs[1].at[pl.ds(0, VREM * D_MODEL)],
                     dst_hbm.at[pl.ds(VFULL * CHUNK * D_MODEL,
                                      VREM * D_MODEL)],
                     osems[1]).wait()


@jax.jit
def _transpose_table(lut_t, rem_rm):
  mesh = plsc.VectorSubcoreMesh(
      core_axis_name="c", subcore_axis_name="s",
      num_cores=NUM_CORES, num_subcores=NUM_SUBCORES)
  return pl.kernel(
      _tp_body,
      out_type=jax.ShapeDtypeStruct((VOCAB * D_MODEL,), jnp.float32),
      mesh=mesh,
      scratch_types=[
          [pltpu.VMEM((D_MODEL, CHUNK), jnp.float32) for _ in range(NBUF)],
          [pltpu.VMEM((CHUNK * D_MODEL,), jnp.float32) for _ in range(NBUF)],
          [pltpu.SemaphoreType.DMA for _ in range(NBUF)],
          [pltpu.SemaphoreType.DMA for _ in range(NBUF)],
      ],
      compiler_params=pltpu.CompilerParams(needs_layout_passes=False),
      name="sc_table_relayout",
  )(lut_t, rem_rm)


def _emb_body(idx_hbm, tab_hbm, out_hbm, idx_v, rowbufs, obufs, gsems, ssems,
              *, units_per_w):
  wid = lax.axis_index("s") * NUM_CORES + lax.axis_index("c")
  base_u = wid * units_per_w

  pltpu.sync_copy(idx_hbm.at[pl.ds(base_u * CHUNK, units_per_w * CHUNK)],
                  idx_v)

  iota = lax.iota(jnp.int32, LANES)

  def tp(b):
    _diag_transpose_2d(iota, rowbufs[b], obufs[b],
                       CHUNK // LANES, D_MODEL // LANES, scale=SCALE)

  def start_gather(k, b):
    pltpu.async_copy(tab_hbm.at[idx_v.at[pl.ds(k * CHUNK, CHUNK)]],
                     rowbufs[b], gsems[b])

  def wait_gather(b):
    pltpu.make_async_copy(tab_hbm.at[idx_v.at[pl.ds(0, CHUNK)]],
                          rowbufs[b], gsems[b]).wait()

  def start_scatter(k, b):
    u = base_u + k
    t = u // SBLK
    sb = u % SBLK
    for dt in range(D_MODEL // 8):
      off = (((t * 8 + dt) * SBLK + sb) * 8) * CHUNK
      pltpu.async_copy(obufs[b].at[pl.ds(dt * 8 * CHUNK, 8 * CHUNK)],
                       out_hbm.at[pl.ds(off, 8 * CHUNK)], ssems[b])

  def wait_scatter(b):
    for _ in range(D_MODEL // 8):
      pltpu.make_async_copy(obufs[b].at[pl.ds(0, 8 * CHUNK)],
                            out_hbm.at[pl.ds(0, 8 * CHUNK)], ssems[b]).wait()

  for b in range(NBUF):
    start_gather(b, b)

  for b in range(NBUF):
    wait_gather(b)
    tp(b)
    start_scatter(b, b)
    start_gather(NBUF + b, b)

  def step(gg, _):
    for b in range(NBUF):
      k = gg * NBUF + b
      wait_gather(b)
      wait_scatter(b)
      tp(b)
      start_scatter(k, b)
      start_gather(k + NBUF, b)
    return 0

  lax.fori_loop(1, units_per_w // NBUF - 1, step, 0)

  for b in range(NBUF):
    k = units_per_w - NBUF + b
    wait_gather(b)
    wait_scatter(b)
    tp(b)
    start_scatter(k, b)
    wait_scatter(b)


@jax.jit
def _emb_lookup(idx_flat, lut_rm):
  n = idx_flat.shape[0]
  units = n // CHUNK
  assert units % (NUM_WORKERS * NBUF) == 0
  units_per_w = units // NUM_WORKERS
  mesh = plsc.VectorSubcoreMesh(
      core_axis_name="c", subcore_axis_name="s",
      num_cores=NUM_CORES, num_subcores=NUM_SUBCORES)
  body = functools.partial(_emb_body, units_per_w=units_per_w)
  return pl.kernel(
      body,
      out_type=jax.ShapeDtypeStruct((n * D_MODEL,), jnp.float32),
      mesh=mesh,
      scratch_types=[
          pltpu.VMEM((units_per_w * CHUNK,), jnp.int32),
          [pltpu.VMEM((CHUNK, D_MODEL), jnp.float32) for _ in range(NBUF)],
          [pltpu.VMEM((CHUNK * D_MODEL,), jnp.float32) for _ in range(NBUF)],
          [pltpu.SemaphoreType.DMA for _ in range(NBUF)],
          [pltpu.SemaphoreType.DMA for _ in range(NBUF)],
      ],
      compiler_params=pltpu.CompilerParams(
          use_tc_tiling_on_sc=False, needs_layout_passes=False),
      name="sc_embedding_lookup",
  )(idx_flat, lut_rm)


def kernel(x, lut):
  idx_flat = x.T.reshape(-1).astype(jnp.int32)
  rem_rm = lut[VFULL * CHUNK:, :].reshape(-1)
  lut_rm = _transpose_table(lut.T, rem_rm).reshape(VOCAB, D_MODEL)
  flat = _emb_lookup(idx_flat, lut_rm)
  out5 = flat.reshape(N_TOK, 8, SBLK, 8, CHUNK)
  return out5.transpose(2, 4, 0, 1, 3).reshape(N_SEQ, N_TOK, D_MODEL)


# speedup vs baseline: 5.1131x; 1.0021x over previous
import functools
import math

import jax
import jax.numpy as jnp
from jax import lax
from jax.experimental import pallas as pl
from jax.experimental.pallas import tpu as pltpu
from jax.experimental.pallas import tpu_sc as plsc

D_MODEL = 64
SCALE = math.sqrt(D_MODEL)
LANES = 16
NUM_CORES = 2
NUM_SUBCORES = 16
NUM_WORKERS = NUM_CORES * NUM_SUBCORES
CHUNK = 128
NBUF = 4
N_TOK = 200
N_SEQ = 4096
SBLK = N_SEQ // CHUNK
VOCAB = 1000000
VFULL = VOCAB // CHUNK
VREM = VOCAB - VFULL * CHUNK
TP_PER_W = VFULL // NUM_WORKERS
TP_EXTRA = VFULL - TP_PER_W * NUM_WORKERS


def _diag_transpose_flat(iota, src_flat, dst_flat, nr, nc, scale=None):
  clen = LANES * nc
  rlen = LANES * nr
  iota_c = iota * clen

  @plsc.parallel_loop(0, nr * nc, unroll=4)
  def blk_body(blk):
    r0 = (blk // nc) * LANES
    c0 = (blk % nc) * LANES
    sg = r0 * clen + c0
    sd = c0 * rlen + r0

    @plsc.parallel_loop(0, LANES, unroll=16)
    def _k(k):
      mk = (iota + k) & (LANES - 1)
      gidx = iota_c + mk + sg
      v = plsc.load_gather(src_flat, [gidx])
      if scale is not None:
        v = v * scale
      sidx = mk * rlen + iota + sd
      plsc.store_scatter(dst_flat, [sidx], v)


def _diag_transpose_2d(iota, src, dst_flat, nr, nc, scale=None):
  rlen = LANES * nr

  @plsc.parallel_loop(0, nr * nc, unroll=4)
  def blk_body(blk):
    r0 = (blk // nc) * LANES
    c0 = (blk % nc) * LANES
    rvec = iota + r0
    sd = c0 * rlen + r0

    @plsc.parallel_loop(0, LANES, unroll=16)
    def _k(k):
      mk = (iota + k) & (LANES - 1)
      v = plsc.load_gather(src, [rvec, mk + c0])
      if scale is not None:
        v = v * scale
      plsc.store_scatter(dst_flat, [mk * rlen + iota + sd], v)


def _tp_body(src_hbm, rem_hbm, dst_hbm, ibufs, obufs, isems, osems):
  wid = lax.axis_index("s") * NUM_CORES + lax.axis_index("c")
  iota = lax.iota(jnp.int32, LANES)
  base = wid * TP_PER_W

  def tp(b):
    _diag_transpose_2d(iota, ibufs[b], obufs[b],
                       D_MODEL // LANES, CHUNK // LANES)

  def start_in(vc, b):
    for dr in range(D_MODEL // 8):
      pltpu.async_copy(
          src_hbm.at[pl.ds(dr * 8, 8), pl.ds(vc * CHUNK, CHUNK)],
          ibufs[b].at[pl.ds(dr * 8, 8), :], isems[b])

  def wait_in(b):
    for _ in range(D_MODEL // 8):
      pltpu.make_async_copy(
          src_hbm.at[pl.ds(0, 8), pl.ds(0, CHUNK)],
          ibufs[b].at[pl.ds(0, 8), :], isems[b]).wait()

  def start_out(vc, b, width=CHUNK):
    pltpu.async_copy(obufs[b].at[pl.ds(0, width * D_MODEL)],
                     dst_hbm.at[pl.ds(vc * CHUNK * D_MODEL, width * D_MODEL)],
                     osems[b])

  def wait_out(b, width=CHUNK):
    pltpu.make_async_copy(obufs[b].at[pl.ds(0, width * D_MODEL)],
                          dst_hbm.at[pl.ds(0, width * D_MODEL)],
                          osems[b]).wait()

  for b in range(NBUF):
    start_in(base + b, b)

  for b in range(NBUF):
    wait_in(b)
    tp(b)
    start_out(base + b, b)
    start_in(base + NBUF + b, b)

  def step(gg, _):
    for b in range(NBUF):
      wait_in(b)
      wait_out(b)
      tp(b)
      start_out(base + gg * NBUF + b, b)
      start_in(base + (gg + 1) * NBUF + b, b)
    return 0

  lax.fori_loop(1, TP_PER_W // NBUF - 1, step, 0)

  for b in range(NBUF):
    wait_in(b)
    wait_out(b)
    tp(b)
    start_out(base + TP_PER_W - NBUF + b, b)
    wait_out(b)

  @pl.when(wid < TP_EXTRA)
  def _extra_full():
    vc = VFULL - TP_EXTRA + wid
    start_in(vc, 0)
    wait_in(0)
    tp(0)
    start_out(vc, 0)
    wait_out(0)

  @pl.when(wid == TP_EXTRA)
  def _rem():
    pltpu.async_copy(rem_hbm, obufs[1].at[pl.ds(0, VREM * D_MODEL)],
                     osems[1]).wait()
    pltpu.async_copy(obufs[1].at[pl.ds(0, VREM * D_MODEL)],
                     dst_hbm.at[pl.ds(VFULL * CHUNK * D_MODEL,
                                      VREM * D_MODEL)],
                     osems[1]).wait()


@jax.jit
def _transpose_table(lut_t, rem_rm):
  mesh = plsc.VectorSubcoreMesh(
      core_axis_name="c", subcore_axis_name="s",
      num_cores=NUM_CORES, num_subcores=NUM_SUBCORES)
  return pl.kernel(
      _tp_body,
      out_type=jax.ShapeDtypeStruct((VOCAB * D_MODEL,), jnp.float32),
      mesh=mesh,
      scratch_types=[
          [pltpu.VMEM((D_MODEL, CHUNK), jnp.float32) for _ in range(NBUF)],
          [pltpu.VMEM((CHUNK * D_MODEL,), jnp.float32) for _ in range(NBUF)],
          [pltpu.SemaphoreType.DMA for _ in range(NBUF)],
          [pltpu.SemaphoreType.DMA for _ in range(NBUF)],
      ],
      compiler_params=pltpu.CompilerParams(needs_layout_passes=False),
      name="sc_table_relayout",
  )(lut_t, rem_rm)


def _emb_body(idx_hbm, tab_hbm, out_hbm, idx_v, rowbufs, obufs, gsems, ssems,
              *, units_per_w):
  wid = lax.axis_index("s") * NUM_CORES + lax.axis_index("c")
  base_u = wid * units_per_w

  pltpu.sync_copy(idx_hbm.at[pl.ds(base_u * CHUNK, units_per_w * CHUNK)],
                  idx_v)

  iota = lax.iota(jnp.int32, LANES)

  def tp(b):
    _diag_transpose_2d(iota, rowbufs[b], obufs[b],
                       CHUNK // LANES, D_MODEL // LANES, scale=SCALE)

  def start_gather(k, b):
    pltpu.async_copy(tab_hbm.at[idx_v.at[pl.ds(k * CHUNK, CHUNK)]],
                     rowbufs[b], gsems[b])

  def wait_gather(b):
    pltpu.make_async_copy(tab_hbm.at[idx_v.at[pl.ds(0, CHUNK)]],
                          rowbufs[b], gsems[b]).wait()

  def start_scatter(k, b):
    u = base_u + k
    t = u // SBLK
    sb = u % SBLK
    for dt in range(D_MODEL // 8):
      off = (((t * 8 + dt) * SBLK + sb) * 8) * CHUNK
      pltpu.async_copy(obufs[b].at[pl.ds(dt * 8 * CHUNK, 8 * CHUNK)],
                       out_hbm.at[pl.ds(off, 8 * CHUNK)], ssems[b])

  def wait_scatter(b):
    for _ in range(D_MODEL // 8):
      pltpu.make_async_copy(obufs[b].at[pl.ds(0, 8 * CHUNK)],
                            out_hbm.at[pl.ds(0, 8 * CHUNK)], ssems[b]).wait()

  for b in range(NBUF):
    start_gather(b, b)

  for b in range(NBUF):
    wait_gather(b)
    tp(b)
    start_scatter(b, b)
    start_gather(NBUF + b, b)

  def step(gg, _):
    for b in range(NBUF):
      k = gg * NBUF + b
      wait_gather(b)
      wait_scatter(b)
      tp(b)
      start_scatter(k, b)
      start_gather(k + NBUF, b)
    return 0

  lax.fori_loop(1, units_per_w // NBUF - 1, step, 0)

  for b in range(NBUF):
    k = units_per_w - NBUF + b
    wait_gather(b)
    wait_scatter(b)
    tp(b)
    start_scatter(k, b)
    wait_scatter(b)


@jax.jit
def _emb_lookup(idx_flat, lut_rm):
  n = idx_flat.shape[0]
  units = n // CHUNK
  assert units % (NUM_WORKERS * NBUF) == 0
  units_per_w = units // NUM_WORKERS
  mesh = plsc.VectorSubcoreMesh(
      core_axis_name="c", subcore_axis_name="s",
      num_cores=NUM_CORES, num_subcores=NUM_SUBCORES)
  body = functools.partial(_emb_body, units_per_w=units_per_w)
  return pl.kernel(
      body,
      out_type=jax.ShapeDtypeStruct((n * D_MODEL,), jnp.float32),
      mesh=mesh,
      scratch_types=[
          pltpu.VMEM((units_per_w * CHUNK,), jnp.int32),
          [pltpu.VMEM((CHUNK, D_MODEL), jnp.float32) for _ in range(NBUF)],
          [pltpu.VMEM((CHUNK * D_MODEL,), jnp.float32) for _ in range(NBUF)],
          [pltpu.SemaphoreType.DMA for _ in range(NBUF)],
          [pltpu.SemaphoreType.DMA for _ in range(NBUF)],
      ],
      compiler_params=pltpu.CompilerParams(
          use_tc_tiling_on_sc=False, needs_layout_passes=False),
      name="sc_embedding_lookup",
  )(idx_flat, lut_rm)


def kernel(x, lut):
  idx_flat = x.T.reshape(-1).astype(jnp.int32)
  rem_rm = lut[VFULL * CHUNK:, :].reshape(-1)
  lut_rm = _transpose_table(lut.T, rem_rm).reshape(VOCAB, D_MODEL)
  flat = _emb_lookup(idx_flat, lut_rm)
  out5 = flat.reshape(N_TOK, 8, SBLK, 8, CHUNK)
  return out5.transpose(2, 4, 0, 1, 3).reshape(N_SEQ, N_TOK, D_MODEL)
